# baseline (speedup 1.0000x reference)
import functools

import jax
import jax.numpy as jnp
from jax import lax
from jax.experimental import pallas as pl
from jax.experimental.pallas import tpu as pltpu
from jax.experimental.pallas import tpu_sc as plsc

N = 10000
E = 320000
D = 128
NC = 2
NS = 16
NW = NC * NS
EW = E // NW
CB = 2000
NV = CB // 16
NP = 10240
STRIPE = NP // NS


def _proj_matmul(x, w8, b8):
    def mm(x_ref, w_ref, b_ref, *o_refs):
        res = lax.dot_general(w_ref[...], x_ref[...],
                              (((0,), (1,)), ((), ())),
                              preferred_element_type=jnp.float32)
        res = res + b_ref[...]
        for k in range(8):
            o_refs[k][...] = res[k]

    return pl.pallas_call(
        mm,
        out_shape=[jax.ShapeDtypeStruct((N,), jnp.float32)] * 8,
    )(x, w8, b8)


def _edge_pass1(p0, p1, p2, p3, p4, p5, p6, p7, row, col):
    mesh = plsc.VectorSubcoreMesh(core_axis_name="c", subcore_axis_name="s")

    @functools.partial(
        pl.kernel,
        mesh=mesh,
        compiler_params=pltpu.CompilerParams(needs_layout_passes=False),
        out_type=(
            jax.ShapeDtypeStruct((E,), jnp.float32),
            jax.ShapeDtypeStruct((E,), jnp.float32),
            jax.ShapeDtypeStruct((E,), jnp.float32),
            jax.ShapeDtypeStruct((NC, NP), jnp.float32),
            jax.ShapeDtypeStruct((NC, NP), jnp.float32),
        ),
        scratch_types=[
            [pltpu.VMEM((N,), jnp.float32)] * 8,
            pltpu.VMEM((CB,), jnp.int32),
            pltpu.VMEM((CB,), jnp.int32),
            pltpu.VMEM((CB,), jnp.float32),
            pltpu.VMEM((CB,), jnp.float32),
            pltpu.VMEM((CB,), jnp.float32),
            pltpu.VMEM((STRIPE,), jnp.float32),
            pltpu.VMEM_SHARED((NP,), jnp.float32),
            pltpu.VMEM_SHARED((NP,), jnp.float32),
        ],
    )
    def k(p0_hbm, p1_hbm, p2_hbm, p3_hbm, p4_hbm, p5_hbm, p6_hbm, p7_hbm,
          row_hbm, col_hbm, ex0_hbm, ex1_hbm, am_hbm, s0p_hbm, s1p_hbm,
          pv, rowv, colv, ex0v, ex1v, amv, zv, s0s, s1s):
        p_hbms = (p0_hbm, p1_hbm, p2_hbm, p3_hbm, p4_hbm, p5_hbm, p6_hbm,
                  p7_hbm)
        cid = lax.axis_index("c")
        sid = lax.axis_index("s")
        wid = sid * NC + cid

        def zbody(i, _):
            zv[pl.ds(i * 16, 16)] = jnp.zeros((16,), jnp.float32)
            return 0
        lax.fori_loop(0, STRIPE // 16, zbody, 0)
        pltpu.sync_copy(zv, s0s.at[pl.ds(sid * STRIPE, STRIPE)])
        pltpu.sync_copy(zv, s1s.at[pl.ds(sid * STRIPE, STRIPE)])

        for kk in range(8):
            pltpu.sync_copy(p_hbms[kk], pv[kk])
        plsc.subcore_barrier()

        base = wid * EW

        def chunk(ci, _):
            off = base + ci * CB
            pltpu.sync_copy(row_hbm.at[pl.ds(off, CB)], rowv)
            pltpu.sync_copy(col_hbm.at[pl.ds(off, CB)], colv)

            @plsc.parallel_loop(0, NV, 1, unroll=8)
            def body(j):
                sl = pl.ds(j * 16, 16)
                r = rowv[sl]
                c = colv[sl]
                a0 = plsc.load_gather(pv[0], [r])
                a1 = plsc.load_gather(pv[1], [r])
                g0 = plsc.load_gather(pv[2], [r])
                g1 = plsc.load_gather(pv[3], [r])
                b0 = plsc.load_gather(pv[4], [c])
                b1 = plsc.load_gather(pv[5], [c])
                h0 = plsc.load_gather(pv[6], [c])
                h1 = plsc.load_gather(pv[7], [c])
                ex0v[sl] = jnp.exp(jnp.maximum(a0 + b0, 0.0))
                ex1v[sl] = jnp.exp(jnp.maximum(a1 + b1, 0.0))
                amv[sl] = jnp.maximum(g0 + h0, 0.0) + jnp.maximum(g1 + h1, 0.0)

            pltpu.sync_copy(ex0v, ex0_hbm.at[pl.ds(off, CB)])
            pltpu.sync_copy(ex1v, ex1_hbm.at[pl.ds(off, CB)])
            pltpu.sync_copy(amv, am_hbm.at[pl.ds(off, CB)])
            pltpu.sync_copy(ex0v, s0s.at[rowv], add=True)
            pltpu.sync_copy(ex1v, s1s.at[rowv], add=True)
            return 0
        lax.fori_loop(0, EW // CB, chunk, 0)

        plsc.subcore_barrier()

        @pl.when(sid == 0)
        def _dump():
            pltpu.sync_copy(s0s, s0p_hbm.at[cid])
            pltpu.sync_copy(s1s, s1p_hbm.at[cid])

    return k(p0, p1, p2, p3, p4, p5, p6, p7, row, col)


def _edge_pass2(s0p, s1p, row, ex0, ex1, mcode):
    mesh = plsc.VectorSubcoreMesh(core_axis_name="c", subcore_axis_name="s")

    @functools.partial(
        pl.kernel,
        mesh=mesh,
        compiler_params=pltpu.CompilerParams(needs_layout_passes=False),
        out_type=jax.ShapeDtypeStruct((E,), jnp.float32),
        scratch_types=[
            pltpu.VMEM((NC, NP), jnp.float32),
            pltpu.VMEM((NP,), jnp.float32),
            pltpu.VMEM((NP,), jnp.float32),
            pltpu.VMEM((CB,), jnp.int32),
            pltpu.VMEM((CB,), jnp.float32),
            pltpu.VMEM((CB,), jnp.float32),
            pltpu.VMEM((CB,), jnp.int32),
            pltpu.VMEM((CB,), jnp.float32),
        ],
    )
    def k(s0p_hbm, s1p_hbm, row_hbm, ex0_hbm, ex1_hbm, mc_hbm, att_hbm,
          t2, s0v, s1v, rowv, ex0v, ex1v, mcv, attv):
        cid = lax.axis_index("c")
        sid = lax.axis_index("s")
        wid = sid * NC + cid

        pltpu.sync_copy(s0p_hbm, t2)

        @plsc.parallel_loop(0, NP // 16, 1, unroll=8)
        def r0(i):
            sl = pl.ds(i * 16, 16)
            s0v[sl] = t2[0, sl] + t2[1, sl]
        pltpu.sync_copy(s1p_hbm, t2)

        @plsc.parallel_loop(0, NP // 16, 1, unroll=8)
        def r1(i):
            sl = pl.ds(i * 16, 16)
            s1v[sl] = t2[0, sl] + t2[1, sl]

        base = wid * EW

        def chunk(ci, _):
            off = base + ci * CB
            pltpu.sync_copy(row_hbm.at[pl.ds(off, CB)], rowv)
            pltpu.sync_copy(ex0_hbm.at[pl.ds(off, CB)], ex0v)
            pltpu.sync_copy(ex1_hbm.at[pl.ds(off, CB)], ex1v)
            pltpu.sync_copy(mc_hbm.at[pl.ds(off, CB)], mcv)

            @plsc.parallel_loop(0, NV, 1, unroll=8)
            def body(j):
                sl = pl.ds(j * 16, 16)
                r = rowv[sl]
                s0 = plsc.load_gather(s0v, [r])
                s1 = plsc.load_gather(s1v, [r])
                mc = mcv[sl]
                t0 = jnp.where((mc & 1) == 0, 0.0, ex0v[sl] / s0)
                t1 = jnp.where((mc & 2) == 0, 0.0, ex1v[sl] / s1)
                attv[sl] = t0 + t1

            pltpu.sync_copy(attv, att_hbm.at[pl.ds(off, CB)])
            return 0
        lax.fori_loop(0, EW // CB, chunk, 0)

    return k(s0p, s1p, row, ex0, ex1, mcode)


def kernel(x, edge_index, actual_amount, Wc, bc, Wa, ba):
    w8 = jnp.concatenate([Wc[:D], Wa[:D], Wc[D:], Wa[D:]], axis=1)
    b8 = jnp.concatenate([bc, ba, jnp.zeros((4,), jnp.float32)]).reshape(8, 1)
    ps = _proj_matmul(x, w8, b8)
    z = (ps[0][0] * 0.0).astype(jnp.int32)
    sel = jnp.where(actual_amount != 0, jnp.array([1, 2], jnp.int32), 0)
    mcode = jnp.sum(sel, axis=1, dtype=jnp.int32) + z
    row = edge_index[0]
    col = edge_index[1]
    ex0, ex1, amount, s0p, s1p = _edge_pass1(*ps, row, col)
    att = _edge_pass2(s0p, s1p, row, ex0, ex1, mcode)
    return (att, amount)


# speedup vs baseline: 1.5533x; 1.5533x over previous
import functools

import jax
import jax.numpy as jnp
from jax import lax
from jax.experimental import pallas as pl
from jax.experimental.pallas import tpu as pltpu
from jax.experimental.pallas import tpu_sc as plsc

N = 10000
E = 320000
D = 128
NC = 2
NS = 16
NW = NC * NS
CB = 2560
NV = CB // 16
NCH = E // CB
KMAX = (NCH + NW - 1) // NW
NP = 10240
STRIPE = NP // NS
UNROLL = 8


def _proj_matmul(x, w8, b8):
    def mm(x_ref, w_ref, b_ref, *o_refs):
        res = lax.dot_general(w_ref[...], x_ref[...],
                              (((0,), (1,)), ((), ())),
                              preferred_element_type=jnp.float32)
        res = res + b_ref[...]
        for k in range(8):
            o_refs[k][...] = res[k]

    return pl.pallas_call(
        mm,
        out_shape=[jax.ShapeDtypeStruct((N,), jnp.float32)] * 8,
    )(x, w8, b8)


def _edge_pass1(p0, p1, p2, p3, p4, p5, p6, p7, edge_index):
    mesh = plsc.VectorSubcoreMesh(core_axis_name="c", subcore_axis_name="s")

    @functools.partial(
        pl.kernel,
        mesh=mesh,
        compiler_params=pltpu.CompilerParams(needs_layout_passes=False),
        out_type=(
            jax.ShapeDtypeStruct((E,), jnp.float32),
            jax.ShapeDtypeStruct((E,), jnp.float32),
            jax.ShapeDtypeStruct((E,), jnp.float32),
            jax.ShapeDtypeStruct((NC, NP), jnp.float32),
            jax.ShapeDtypeStruct((NC, NP), jnp.float32),
        ),
        scratch_types=[
            [pltpu.VMEM((N,), jnp.float32)] * 8,
            [pltpu.VMEM((2, CB), jnp.int32)] * 2,
            [pltpu.VMEM((CB,), jnp.int32)] * 2,
            [pltpu.VMEM((CB,), jnp.float32)] * 2,
            [pltpu.VMEM((CB,), jnp.float32)] * 2,
            [pltpu.VMEM((CB,), jnp.float32)] * 2,
            pltpu.VMEM((STRIPE,), jnp.float32),
            pltpu.VMEM_SHARED((NP,), jnp.float32),
            pltpu.VMEM_SHARED((NP,), jnp.float32),
            pltpu.SemaphoreType.DMA,
            pltpu.SemaphoreType.DMA,
            pltpu.SemaphoreType.DMA,
            pltpu.SemaphoreType.DMA,
            pltpu.SemaphoreType.DMA,
        ],
    )
    def k(p0_hbm, p1_hbm, p2_hbm, p3_hbm, p4_hbm, p5_hbm, p6_hbm, p7_hbm,
          ei_hbm, ex0_hbm, ex1_hbm, am_hbm, s0p_hbm, s1p_hbm,
          pv, eiv, rowv, ex0v, ex1v, amv, zv, s0s, s1s,
          psem, isem0, isem1, osem0, osem1):
        p_hbms = (p0_hbm, p1_hbm, p2_hbm, p3_hbm, p4_hbm, p5_hbm, p6_hbm,
                  p7_hbm)
        isems = (isem0, isem1)
        osems = (osem0, osem1)
        cid = lax.axis_index("c")
        sid = lax.axis_index("s")
        wid = sid * NC + cid

        pcopies = [pltpu.async_copy(p_hbms[kk], pv[kk], psem)
                   for kk in range(8)]

        @plsc.parallel_loop(0, STRIPE // 16, 1, unroll=UNROLL)
        def zbody(i):
            zv[pl.ds(i * 16, 16)] = jnp.zeros((16,), jnp.float32)
        pltpu.sync_copy(zv, s0s.at[pl.ds(sid * STRIPE, STRIPE)])
        pltpu.sync_copy(zv, s1s.at[pl.ds(sid * STRIPE, STRIPE)])

        pltpu.async_copy(ei_hbm.at[:, pl.ds(wid * CB, CB)], eiv[0], isems[0])

        for d in pcopies:
            d.wait()
        plsc.subcore_barrier()

        def in_wait(buf):
            pltpu.make_async_copy(
                ei_hbm.at[:, pl.ds(0, CB)], eiv[buf], isems[buf]).wait()

        def out_wait(buf):
            pltpu.make_async_copy(
                ex0v[buf], ex0_hbm.at[pl.ds(0, CB)], osems[buf]).wait()
            pltpu.make_async_copy(
                ex1v[buf], ex1_hbm.at[pl.ds(0, CB)], osems[buf]).wait()
            pltpu.make_async_copy(
                amv[buf], am_hbm.at[pl.ds(0, CB)], osems[buf]).wait()

        for kk in range(KMAX):
            c = wid + NW * kk
            buf = kk % 2

            @pl.when(c < NCH)
            def _chunk():
                off = c * CB
                in_wait(buf)
                if kk >= 1:
                    @pl.when((c - NW) >= 0)
                    def _():
                        out_wait(1 - buf)
                if kk + 1 < KMAX:
                    @pl.when((c + NW) < NCH)
                    def _():
                        pltpu.async_copy(
                            ei_hbm.at[:, pl.ds((c + NW) * CB, CB)],
                            eiv[1 - buf], isems[1 - buf])

                @plsc.parallel_loop(0, NV, 1, unroll=UNROLL)
                def body(j):
                    sl = pl.ds(j * 16, 16)
                    r = eiv[buf][0, sl]
                    cc = eiv[buf][1, sl]
                    rowv[buf][sl] = r
                    a0 = plsc.load_gather(pv[0], [r])
                    a1 = plsc.load_gather(pv[1], [r])
                    g0 = plsc.load_gather(pv[2], [r])
                    g1 = plsc.load_gather(pv[3], [r])
                    b0 = plsc.load_gather(pv[4], [cc])
                    b1 = plsc.load_gather(pv[5], [cc])
                    h0 = plsc.load_gather(pv[6], [cc])
                    h1 = plsc.load_gather(pv[7], [cc])
                    ex0v[buf][sl] = jnp.exp(jnp.maximum(a0 + b0, 0.0))
                    ex1v[buf][sl] = jnp.exp(jnp.maximum(a1 + b1, 0.0))
                    amv[buf][sl] = (jnp.maximum(g0 + h0, 0.0)
                                    + jnp.maximum(g1 + h1, 0.0))

                pltpu.async_copy(ex0v[buf], ex0_hbm.at[pl.ds(off, CB)],
                                 osems[buf])
                pltpu.async_copy(ex1v[buf], ex1_hbm.at[pl.ds(off, CB)],
                                 osems[buf])
                pltpu.async_copy(amv[buf], am_hbm.at[pl.ds(off, CB)],
                                 osems[buf])
                pltpu.sync_copy(ex0v[buf], s0s.at[rowv[buf]], add=True)
                pltpu.sync_copy(ex1v[buf], s1s.at[rowv[buf]], add=True)

        c_last = wid + NW * (KMAX - 1)

        @pl.when(c_last < NCH)
        def _():
            out_wait((KMAX - 1) % 2)

        @pl.when(c_last >= NCH)
        def _():
            out_wait((KMAX - 2) % 2)

        plsc.subcore_barrier()

        @pl.when(sid == 0)
        def _dump():
            pltpu.sync_copy(s0s, s0p_hbm.at[cid])
            pltpu.sync_copy(s1s, s1p_hbm.at[cid])

    return k(p0, p1, p2, p3, p4, p5, p6, p7, edge_index)


def _edge_pass2(s0p, s1p, edge_index, ex0, ex1, mcode):
    mesh = plsc.VectorSubcoreMesh(core_axis_name="c", subcore_axis_name="s")

    @functools.partial(
        pl.kernel,
        mesh=mesh,
        compiler_params=pltpu.CompilerParams(needs_layout_passes=False),
        out_type=jax.ShapeDtypeStruct((E,), jnp.float32),
        scratch_types=[
            pltpu.VMEM((2, NP), jnp.float32),
            pltpu.VMEM((2, NP), jnp.float32),
            pltpu.VMEM((NP,), jnp.float32),
            pltpu.VMEM((NP,), jnp.float32),
            [pltpu.VMEM((2, CB), jnp.int32)] * 2,
            [pltpu.VMEM((CB,), jnp.float32)] * 2,
            [pltpu.VMEM((CB,), jnp.float32)] * 2,
            [pltpu.VMEM((CB,), jnp.int32)] * 2,
            [pltpu.VMEM((CB,), jnp.float32)] * 2,
            pltpu.SemaphoreType.DMA,
            pltpu.SemaphoreType.DMA,
            pltpu.SemaphoreType.DMA,
            pltpu.SemaphoreType.DMA,
            pltpu.SemaphoreType.DMA,
            pltpu.SemaphoreType.DMA,
        ],
    )
    def k(s0p_hbm, s1p_hbm, ei_hbm, ex0_hbm, ex1_hbm, mc_hbm, att_hbm,
          t2a, t2b, s0v, s1v, eiv, ex0v, ex1v, mcv, attv,
          tsem, tsem2, isem0, isem1, osem0, osem1):
        isems = (isem0, isem1)
        osems = (osem0, osem1)
        cid = lax.axis_index("c")
        sid = lax.axis_index("s")
        wid = sid * NC + cid

        ca = pltpu.async_copy(s0p_hbm, t2a, tsem)
        cb = pltpu.async_copy(s1p_hbm, t2b, tsem2)
        pltpu.async_copy(ei_hbm.at[:, pl.ds(wid * CB, CB)], eiv[0], isems[0])
        pltpu.async_copy(ex0_hbm.at[pl.ds(wid * CB, CB)], ex0v[0], isems[0])
        pltpu.async_copy(ex1_hbm.at[pl.ds(wid * CB, CB)], ex1v[0], isems[0])
        pltpu.async_copy(mc_hbm.at[pl.ds(wid * CB, CB)], mcv[0], isems[0])

        ca.wait()

        @plsc.parallel_loop(0, NP // 16, 1, unroll=UNROLL)
        def r0(i):
            sl = pl.ds(i * 16, 16)
            s0v[sl] = t2a[0, sl] + t2a[1, sl]
        cb.wait()

        @plsc.parallel_loop(0, NP // 16, 1, unroll=UNROLL)
        def r1(i):
            sl = pl.ds(i * 16, 16)
            s1v[sl] = t2b[0, sl] + t2b[1, sl]

        def in_wait(buf):
            pltpu.make_async_copy(
                ei_hbm.at[:, pl.ds(0, CB)], eiv[buf], isems[buf]).wait()
            pltpu.make_async_copy(
                ex0_hbm.at[pl.ds(0, CB)], ex0v[buf], isems[buf]).wait()
            pltpu.make_async_copy(
                ex1_hbm.at[pl.ds(0, CB)], ex1v[buf], isems[buf]).wait()
            pltpu.make_async_copy(
                mc_hbm.at[pl.ds(0, CB)], mcv[buf], isems[buf]).wait()

        def in_start(c, buf):
            off = c * CB
            pltpu.async_copy(ei_hbm.at[:, pl.ds(off, CB)], eiv[buf],
                             isems[buf])
            pltpu.async_copy(ex0_hbm.at[pl.ds(off, CB)], ex0v[buf],
                             isems[buf])
            pltpu.async_copy(ex1_hbm.at[pl.ds(off, CB)], ex1v[buf],
                             isems[buf])
            pltpu.async_copy(mc_hbm.at[pl.ds(off, CB)], mcv[buf], isems[buf])

        def out_wait(buf):
            pltpu.make_async_copy(
                attv[buf], att_hbm.at[pl.ds(0, CB)], osems[buf]).wait()

        for kk in range(KMAX):
            c = wid + NW * kk
            buf = kk % 2

            @pl.when(c < NCH)
            def _chunk():
                off = c * CB
                in_wait(buf)
                if kk >= 2:
                    @pl.when((c - 2 * NW) >= 0)
                    def _():
                        out_wait(buf)
                if kk + 1 < KMAX:
                    @pl.when((c + NW) < NCH)
                    def _():
                        in_start(c + NW, 1 - buf)

                @plsc.parallel_loop(0, NV, 1, unroll=UNROLL)
                def body(j):
                    sl = pl.ds(j * 16, 16)
                    r = eiv[buf][0, sl]
                    s0 = plsc.load_gather(s0v, [r])
                    s1 = plsc.load_gather(s1v, [r])
                    mc = mcv[buf][sl]
                    t0 = jnp.where((mc & 1) == 0, 0.0, ex0v[buf][sl] / s0)
                    t1 = jnp.where((mc & 2) == 0, 0.0, ex1v[buf][sl] / s1)
                    attv[buf][sl] = t0 + t1

                pltpu.async_copy(attv[buf], att_hbm.at[pl.ds(off, CB)],
                                 osems[buf])

        out_wait(0)
        out_wait(1)

    return k(s0p, s1p, edge_index, ex0, ex1, mcode)


def kernel(x, edge_index, actual_amount, Wc, bc, Wa, ba):
    w8 = jnp.concatenate([Wc[:D], Wa[:D], Wc[D:], Wa[D:]], axis=1)
    b8 = jnp.concatenate([bc, ba, jnp.zeros((4,), jnp.float32)]).reshape(8, 1)
    ps = _proj_matmul(x, w8, b8)
    aa_b, _ = lax.optimization_barrier((actual_amount, ps[0]))
    sel = jnp.where(aa_b != 0, jnp.array([1, 2], jnp.int32), 0)
    mcode = jnp.sum(sel, axis=1, dtype=jnp.int32)
    ex0, ex1, amount, s0p, s1p = _edge_pass1(*ps, edge_index)
    att = _edge_pass2(s0p, s1p, edge_index, ex0, ex1, mcode)
    return (att, amount)
